# Initial kernel scaffold; baseline (speedup 1.0000x reference)
import jax
import jax.numpy as jnp
from jax.experimental import pallas as pl


def kernel(source_points, target_points, source_density, target_density):
    raise NotImplementedError("write your pallas kernel here")


# speedup vs baseline: 1.5337x; 1.5337x over previous
import functools

import jax
import jax.numpy as jnp
import numpy as np
from jax import lax
from jax.experimental import pallas as pl
from jax.experimental.pallas import tpu as pltpu

_RADIUS = 0.08
_K = 64
_NEG_INF = jnp.inf


def _topk_body(tgt_ref, src_ref, dist_ref, idx_ref, stats_ref):
    tq = tgt_ref[0]
    s = src_ref[0]
    T = tq.shape[0]
    Ns = s.shape[1]
    r0 = jnp.sum(tq * tq, axis=1, keepdims=True)
    r1 = jnp.sum(s * s, axis=0, keepdims=True)
    m = jnp.dot(tq[:, 0:3], s, preferred_element_type=jnp.float32)
    d = r0 - 2.0 * m + r1

    iota_j = lax.broadcasted_iota(jnp.int32, (T, Ns), 1)
    work = d
    dcols = []
    icols = []
    for k in range(_K):
        mn = jnp.min(work, axis=1, keepdims=True)
        cand = jnp.where(work == mn, iota_j, jnp.int32(2**30))
        am = jnp.min(cand, axis=1, keepdims=True)
        dcols.append(mn)
        icols.append(am)
        work = jnp.where(iota_j == am, _NEG_INF, work)
    dvals = jnp.concatenate(dcols, axis=1)
    ivals = jnp.concatenate(icols, axis=1)

    rad2 = jnp.float32(_RADIUS) ** 2
    maskv = rad2 >= dvals
    pidx = jnp.where(maskv, ivals, jnp.int32(-1))
    dist = jnp.sqrt(jnp.maximum(dvals, 1e-9)) / jnp.float32(_RADIUS + 1e-6)
    cnt = jnp.sum((pidx < 0).astype(jnp.float32), axis=1, keepdims=True)
    psize = jnp.float32(_K) - cnt
    stats = jnp.concatenate([cnt, psize] + [cnt] * 6, axis=1)

    dist_ref[0] = dist
    idx_ref[0] = pidx
    stats_ref[0] = stats


def _run_topk(tgt_pad, src_t, interpret=False):
    B, Nt, _ = tgt_pad.shape
    Ns = src_t.shape[2]
    T = min(128, Nt)
    grid = (B, Nt // T)
    out_shapes = (
        jax.ShapeDtypeStruct((B, Nt, _K), jnp.float32),
        jax.ShapeDtypeStruct((B, Nt, _K), jnp.int32),
        jax.ShapeDtypeStruct((B, Nt, 8), jnp.float32),
    )
    return pl.pallas_call(
        _topk_body,
        grid=grid,
        in_specs=[
            pl.BlockSpec((1, T, 8), lambda b, i: (b, i, 0)),
            pl.BlockSpec((1, 3, Ns), lambda b, i: (b, 0, 0)),
        ],
        out_specs=(
            pl.BlockSpec((1, T, _K), lambda b, i: (b, i, 0)),
            pl.BlockSpec((1, T, _K), lambda b, i: (b, i, 0)),
            pl.BlockSpec((1, T, 8), lambda b, i: (b, i, 0)),
        ),
        out_shape=out_shapes,
        interpret=interpret,
    )(tgt_pad, src_t)


def kernel(source_points, target_points, source_density, target_density):
    B, Ns, _ = source_points.shape
    _, Nt, _ = target_points.shape
    rad = jnp.full((B, 1, 1), _RADIUS, dtype=jnp.float32)

    tgt_pad = jnp.concatenate(
        [target_points, jnp.zeros((B, Nt, 5), jnp.float32)], axis=2)
    src_t = source_points.transpose(0, 2, 1)
    dist, pidx, stats = _run_topk(tgt_pad, src_t)

    mask_cnt = stats[:, :, 0:1]
    patches_size = stats[:, :, 1]

    src_n = source_points / (rad + 1e-06)
    tgt_n = target_points / (rad + 1e-06)
    bidx = jnp.arange(B, dtype=jnp.int32).reshape(B, 1, 1)
    bidx = jnp.broadcast_to(bidx, (B, Nt, _K))
    safe = jnp.maximum(pidx, 0)
    patches = src_n[bidx, safe, :]
    patches = jnp.where((pidx < 0)[..., None], 0.0, patches)
    patches = patches - tgt_n[:, :, None, :]
    density = source_density[bidx, safe, :]
    density = jnp.where((pidx < 0)[..., None], 0.0, density)
    weight_density = jnp.sum(density, axis=-2) / mask_cnt

    idx = jnp.stack([bidx, pidx], axis=-1)
    H = int(round(float(np.cbrt(Nt))))
    out_patches = patches.reshape(B, H, H, H, _K, 3)
    out_idx = idx.reshape(B, H, H, H, _K, 2)
    out_dist = dist.reshape(B, H, H, H, _K)
    return (out_patches, out_idx, out_dist, patches_size, rad, weight_density)


# speedup vs baseline: 1.7730x; 1.1560x over previous
import functools

import jax
import jax.numpy as jnp
import numpy as np
from jax import lax
from jax.experimental import pallas as pl
from jax.experimental.pallas import tpu as pltpu
from jax.experimental.pallas import tpu_sc as plsc

_RADIUS = 0.08
_K = 64
_NEG_INF = jnp.inf


def _flip_pow2(x, axis):
    n = x.shape[axis]
    pre, post = x.shape[:axis], x.shape[axis + 1:]
    p = n // 2
    while p >= 1:
        x = x.reshape(*pre, n // (2 * p), 2, p, *post)
        sl = (slice(None),) * (len(pre) + 1)
        x = jnp.concatenate([x[sl + (slice(1, 2),)], x[sl + (slice(0, 1),)]],
                            axis=len(pre) + 1)
        x = x.reshape(*pre, n, *post)
        p //= 2
    return x


def _bitonic_merge(key, idx, block, G, L, T):
    p = block // 2
    while p >= 1:
        kk = key.reshape(G // (2 * p), 2, p, L, T)
        ii = idx.reshape(G // (2 * p), 2, p, L, T)
        lo_k, hi_k = kk[:, 0], kk[:, 1]
        lo_i, hi_i = ii[:, 0], ii[:, 1]
        c = lo_k <= hi_k
        key = jnp.stack([jnp.minimum(lo_k, hi_k), jnp.maximum(lo_k, hi_k)],
                        axis=1).reshape(G, L, T)
        idx = jnp.stack([jnp.where(c, lo_i, hi_i), jnp.where(c, hi_i, lo_i)],
                        axis=1).reshape(G, L, T)
        p //= 2
    return key, idx


def _sort_cols(key, idx, G, L, T):
    s = 1
    while s < G:
        kk = key.reshape(G // (2 * s), 2, s, L, T)
        ii = idx.reshape(G // (2 * s), 2, s, L, T)
        key = jnp.concatenate([kk[:, 0], _flip_pow2(kk[:, 1], 1)],
                              axis=1).reshape(G, L, T)
        idx = jnp.concatenate([ii[:, 0], _flip_pow2(ii[:, 1], 1)],
                              axis=1).reshape(G, L, T)
        key, idx = _bitonic_merge(key, idx, 2 * s, G, L, T)
        s *= 2
    return key, idx


def _keep_min(a_k, a_i, b_k, b_i, G, L, T):
    b_k = _flip_pow2(b_k, 0)
    b_i = _flip_pow2(b_i, 0)
    c = a_k <= b_k
    k = jnp.minimum(a_k, b_k)
    i = jnp.where(c, a_i, b_i)
    return _bitonic_merge(k, i, G, G, L, T)


def _chunk_top64(d, T, NC, base):
    G = _K
    L = NC // G
    key = d.reshape(G, L, T)
    idx = (lax.broadcasted_iota(jnp.int32, (G, L, T), 0) * L
           + lax.broadcasted_iota(jnp.int32, (G, L, T), 1) + base)
    key, idx = _sort_cols(key, idx, G, L, T)
    while L > 1:
        h = L // 2
        key, idx = _keep_min(key[:, :h], idx[:, :h],
                             key[:, h:], idx[:, h:], G, h, T)
        L = h
    return key, idx


def _topk_body(src_ref, tgt_ref, dist_ref, idx_ref, stats_ref):
    sq_ = src_ref[0]
    tt = tgt_ref[0]
    Ns = sq_.shape[0]
    T = tt.shape[1]
    r1 = jnp.sum(sq_ * sq_, axis=1, keepdims=True)
    r0 = jnp.sum(tt * tt, axis=0, keepdims=True)

    NC = min(_NC, Ns)
    r_key = r_idx = None
    for ci in range(Ns // NC):
        sc = sq_[ci * NC:(ci + 1) * NC, 0:3]
        m = jnp.dot(sc, tt, preferred_element_type=jnp.float32)
        d = r0 - 2.0 * m + r1[ci * NC:(ci + 1) * NC]
        c_key, c_idx = _chunk_top64(d, T, NC, ci * NC)
        if r_key is None:
            r_key, r_idx = c_key, c_idx
        else:
            r_key, r_idx = _keep_min(r_key, r_idx, c_key, c_idx, _K, 1, T)
    dvals, ivals = r_key[:, 0], r_idx[:, 0]

    rad2 = jnp.float32(_RADIUS) ** 2
    maskv = rad2 >= dvals
    pidx = jnp.where(maskv, ivals, jnp.int32(-1))
    dist = jnp.sqrt(jnp.maximum(dvals, 1e-9)) / jnp.float32(_RADIUS + 1e-6)
    cnt = jnp.sum((pidx < 0).astype(jnp.float32), axis=0, keepdims=True)
    psize = jnp.float32(_K) - cnt
    stats = jnp.concatenate([cnt, psize] + [cnt] * 6, axis=0)

    dist_ref[0] = dist
    idx_ref[0] = pidx
    stats_ref[0] = stats


_NC = 512


def _run_topk(src_pad, tgt_t, interpret=False):
    B, Ns, _ = src_pad.shape
    Nt = tgt_t.shape[2]
    T = min(128, Nt)
    grid = (B, Nt // T)
    out_shapes = (
        jax.ShapeDtypeStruct((B, _K, Nt), jnp.float32),
        jax.ShapeDtypeStruct((B, _K, Nt), jnp.int32),
        jax.ShapeDtypeStruct((B, 8, Nt), jnp.float32),
    )
    return pl.pallas_call(
        _topk_body,
        grid=grid,
        in_specs=[
            pl.BlockSpec((1, Ns, 8), lambda b, i: (b, 0, 0)),
            pl.BlockSpec((1, 3, T), lambda b, i: (b, 0, i)),
        ],
        out_specs=(
            pl.BlockSpec((1, _K, T), lambda b, i: (b, 0, i)),
            pl.BlockSpec((1, _K, T), lambda b, i: (b, 0, i)),
            pl.BlockSpec((1, 8, T), lambda b, i: (b, 0, i)),
        ),
        out_shape=out_shapes,
        interpret=interpret,
    )(src_pad, tgt_t)


def _sc_gather(planes, idx_flat, tgt_planes, B, Nt, Ns):
    NW = 32
    ENT = B * Nt * _K
    epw = ENT // NW
    rpw = epw // _K
    wpb = NW // B

    mesh = plsc.VectorSubcoreMesh(core_axis_name="c", subcore_axis_name="s")

    @functools.partial(
        pl.kernel,
        out_type=(
            jax.ShapeDtypeStruct((ENT,), jnp.float32),
            jax.ShapeDtypeStruct((ENT,), jnp.float32),
            jax.ShapeDtypeStruct((ENT,), jnp.float32),
            jax.ShapeDtypeStruct((B * Nt,), jnp.float32),
        ),
        mesh=mesh,
        scratch_types=[
            pltpu.VMEM((Ns,), jnp.float32),
            pltpu.VMEM((Ns,), jnp.float32),
            pltpu.VMEM((Ns,), jnp.float32),
            pltpu.VMEM((Ns,), jnp.float32),
            pltpu.VMEM((epw,), jnp.int32),
            pltpu.VMEM((epw,), jnp.float32),
            pltpu.VMEM((epw,), jnp.float32),
            pltpu.VMEM((epw,), jnp.float32),
            pltpu.VMEM((rpw,), jnp.float32),
            pltpu.VMEM((rpw,), jnp.float32),
            pltpu.VMEM((rpw,), jnp.float32),
            pltpu.VMEM((rpw,), jnp.float32),
        ],
    )
    def k(planes_hbm, idx_hbm, tgt_hbm, ox_hbm, oy_hbm, oz_hbm, ds_hbm,
          px, py, pz, pd, idx_v, ox_v, oy_v, oz_v, ds_v, tx_v, ty_v, tz_v):
        wid = lax.axis_index("c") * (NW // 2) + lax.axis_index("s")
        b = wid // wpb
        base_e = wid * epw
        base_r = wid * rpw
        row_in_b = (wid % wpb) * rpw
        pltpu.sync_copy(planes_hbm.at[pl.ds((b * 4 + 0) * Ns, Ns)], px)
        pltpu.sync_copy(planes_hbm.at[pl.ds((b * 4 + 1) * Ns, Ns)], py)
        pltpu.sync_copy(planes_hbm.at[pl.ds((b * 4 + 2) * Ns, Ns)], pz)
        pltpu.sync_copy(planes_hbm.at[pl.ds((b * 4 + 3) * Ns, Ns)], pd)
        pltpu.sync_copy(idx_hbm.at[pl.ds(base_e, epw)], idx_v)
        pltpu.sync_copy(tgt_hbm.at[pl.ds((b * 3 + 0) * Nt + row_in_b, rpw)], tx_v)
        pltpu.sync_copy(tgt_hbm.at[pl.ds((b * 3 + 1) * Nt + row_in_b, rpw)], ty_v)
        pltpu.sync_copy(tgt_hbm.at[pl.ds((b * 3 + 2) * Nt + row_in_b, rpw)], tz_v)

        lanes = lax.iota(jnp.int32, 16)

        def block_body(rb, carry):
            tx = tx_v[pl.ds(rb * 16, 16)]
            ty = ty_v[pl.ds(rb * 16, 16)]
            tz = tz_v[pl.ds(rb * 16, 16)]
            base = rb * (16 * _K) + lanes * _K
            acc = jnp.zeros((16,), jnp.float32)
            for u in range(_K):
                pos = base + u
                iv = plsc.load_gather(idx_v, [pos])
                msk = iv >= 0
                safe = jnp.where(msk, iv, 0)
                gx = jnp.where(msk, plsc.load_gather(px, [safe]), 0.0)
                gy = jnp.where(msk, plsc.load_gather(py, [safe]), 0.0)
                gz = jnp.where(msk, plsc.load_gather(pz, [safe]), 0.0)
                gd = jnp.where(msk, plsc.load_gather(pd, [safe]), 0.0)
                plsc.store_scatter(ox_v, [pos], gx - tx)
                plsc.store_scatter(oy_v, [pos], gy - ty)
                plsc.store_scatter(oz_v, [pos], gz - tz)
                acc = acc + gd
            ds_v[pl.ds(rb * 16, 16)] = acc
            return carry

        lax.fori_loop(0, rpw // 16, block_body, jnp.int32(0))

        pltpu.sync_copy(ox_v, ox_hbm.at[pl.ds(base_e, epw)])
        pltpu.sync_copy(oy_v, oy_hbm.at[pl.ds(base_e, epw)])
        pltpu.sync_copy(oz_v, oz_hbm.at[pl.ds(base_e, epw)])
        pltpu.sync_copy(ds_v, ds_hbm.at[pl.ds(base_r, rpw)])

    return k(planes, idx_flat, tgt_planes)


def kernel(source_points, target_points, source_density, target_density):
    B, Ns, _ = source_points.shape
    _, Nt, _ = target_points.shape
    rad = jnp.full((B, 1, 1), _RADIUS, dtype=jnp.float32)

    src_pad = jnp.concatenate(
        [source_points, jnp.zeros((B, Ns, 5), jnp.float32)], axis=2)
    tgt_t = target_points.transpose(0, 2, 1)
    dist_t, pidx_t, stats_t = _run_topk(src_pad, tgt_t)
    dist = dist_t.transpose(0, 2, 1)
    pidx = pidx_t.transpose(0, 2, 1)

    mask_cnt = stats_t[:, 0, :, None]
    patches_size = stats_t[:, 1, :]

    src_n = source_points / (rad + 1e-06)
    tgt_n = target_points / (rad + 1e-06)
    bidx = jnp.arange(B, dtype=jnp.int32).reshape(B, 1, 1)
    bidx = jnp.broadcast_to(bidx, (B, Nt, _K))
    safe = jnp.maximum(pidx, 0)
    patches = src_n[bidx, safe, :]
    patches = jnp.where((pidx < 0)[..., None], 0.0, patches)
    patches = patches - tgt_n[:, :, None, :]
    density = source_density[bidx, safe, :]
    density = jnp.where((pidx < 0)[..., None], 0.0, density)
    weight_density = jnp.sum(density, axis=-2) / mask_cnt

    idx = jnp.stack([bidx, pidx], axis=-1)
    H = int(round(float(np.cbrt(Nt))))
    out_patches = patches.reshape(B, H, H, H, _K, 3)
    out_idx = idx.reshape(B, H, H, H, _K, 2)
    out_dist = dist.reshape(B, H, H, H, _K)
    return (out_patches, out_idx, out_dist, patches_size, rad, weight_density)


# speedup vs baseline: 9.8200x; 5.5386x over previous
import jax
import jax.numpy as jnp
import numpy as np
from jax import lax
from jax.experimental import pallas as pl

_RADIUS = 0.08
_K = 64
_NEG_INF = jnp.inf


def _flip_pow2(x, axis):
    n = x.shape[axis]
    pre, post = x.shape[:axis], x.shape[axis + 1:]
    p = n // 2
    while p >= 1:
        x = x.reshape(*pre, n // (2 * p), 2, p, *post)
        sl = (slice(None),) * (len(pre) + 1)
        x = jnp.concatenate([x[sl + (slice(1, 2),)], x[sl + (slice(0, 1),)]],
                            axis=len(pre) + 1)
        x = x.reshape(*pre, n, *post)
        p //= 2
    return x


def _bitonic_merge(key, idx, block, G, L, T):
    p = block // 2
    while p >= 1:
        kk = key.reshape(G // (2 * p), 2, p, L, T)
        ii = idx.reshape(G // (2 * p), 2, p, L, T)
        lo_k, hi_k = kk[:, 0], kk[:, 1]
        lo_i, hi_i = ii[:, 0], ii[:, 1]
        c = lo_k <= hi_k
        key = jnp.stack([jnp.minimum(lo_k, hi_k), jnp.maximum(lo_k, hi_k)],
                        axis=1).reshape(G, L, T)
        idx = jnp.stack([jnp.where(c, lo_i, hi_i), jnp.where(c, hi_i, lo_i)],
                        axis=1).reshape(G, L, T)
        p //= 2
    return key, idx


def _sort_cols(key, idx, G, L, T):
    s = 1
    while s < G:
        kk = key.reshape(G // (2 * s), 2, s, L, T)
        ii = idx.reshape(G // (2 * s), 2, s, L, T)
        key = jnp.concatenate([kk[:, 0], _flip_pow2(kk[:, 1], 1)],
                              axis=1).reshape(G, L, T)
        idx = jnp.concatenate([ii[:, 0], _flip_pow2(ii[:, 1], 1)],
                              axis=1).reshape(G, L, T)
        key, idx = _bitonic_merge(key, idx, 2 * s, G, L, T)
        s *= 2
    return key, idx


def _keep_min(a_k, a_i, b_k, b_i, G, L, T):
    b_k = _flip_pow2(b_k, 0)
    b_i = _flip_pow2(b_i, 0)
    c = a_k <= b_k
    k = jnp.minimum(a_k, b_k)
    i = jnp.where(c, a_i, b_i)
    return _bitonic_merge(k, i, G, G, L, T)


def _chunk_top64(d, T, NC, base):
    G = _K
    L = NC // G
    key = d.reshape(G, L, T)
    idx = (lax.broadcasted_iota(jnp.int32, (G, L, T), 0) * L
           + lax.broadcasted_iota(jnp.int32, (G, L, T), 1) + base)
    key, idx = _sort_cols(key, idx, G, L, T)
    while L > 1:
        h = L // 2
        key, idx = _keep_min(key[:, :h], idx[:, :h],
                             key[:, h:], idx[:, h:], G, h, T)
        L = h
    return key, idx


def _topk_body(src_ref, tgt_ref, dist_ref, idx_ref, gidx_ref, stats_ref):
    sq_ = src_ref[0]
    tt = tgt_ref[0]
    Ns = sq_.shape[0]
    T = tt.shape[1]
    r1 = jnp.sum(sq_ * sq_, axis=1, keepdims=True)
    r0 = jnp.sum(tt * tt, axis=0, keepdims=True)

    NC = min(_NC, Ns)
    r_key = r_idx = None
    for ci in range(Ns // NC):
        sc = sq_[ci * NC:(ci + 1) * NC, 0:3]
        m = jnp.dot(sc, tt, preferred_element_type=jnp.float32)
        d = r0 - 2.0 * m + r1[ci * NC:(ci + 1) * NC]
        c_key, c_idx = _chunk_top64(d, T, NC, ci * NC)
        if r_key is None:
            r_key, r_idx = c_key, c_idx
        else:
            r_key, r_idx = _keep_min(r_key, r_idx, c_key, c_idx, _K, 1, T)
    dvals, ivals = r_key[:, 0], r_idx[:, 0]

    rad2 = jnp.float32(_RADIUS) ** 2
    maskv = rad2 >= dvals
    pidx = jnp.where(maskv, ivals, jnp.int32(-1))
    dist = jnp.sqrt(jnp.maximum(dvals, 1e-9)) / jnp.float32(_RADIUS + 1e-6)
    cnt = jnp.sum((pidx < 0).astype(jnp.float32), axis=0, keepdims=True)
    psize = jnp.float32(_K) - cnt
    stats = jnp.concatenate([cnt, psize] + [cnt] * 6, axis=0)

    b = pl.program_id(0)
    gidx = jnp.where(maskv, b * Ns + ivals,
                     pl.num_programs(0) * Ns + (ivals & (_ZPAD - 1)))

    dist_ref[0] = dist
    idx_ref[0] = pidx
    gidx_ref[0] = gidx
    stats_ref[0] = stats


_NC = 512
_ZPAD = 512
_D = 16


def _run_topk(src_pad, tgt_t, interpret=False):
    B, Ns, _ = src_pad.shape
    Nt = tgt_t.shape[2]
    T = min(128, Nt)
    grid = (B, Nt // T)
    out_shapes = (
        jax.ShapeDtypeStruct((B, _K, Nt), jnp.float32),
        jax.ShapeDtypeStruct((B, _K, Nt), jnp.int32),
        jax.ShapeDtypeStruct((B, _K, Nt), jnp.int32),
        jax.ShapeDtypeStruct((B, 8, Nt), jnp.float32),
    )
    return pl.pallas_call(
        _topk_body,
        grid=grid,
        in_specs=[
            pl.BlockSpec((1, Ns, 8), lambda b, i: (b, 0, 0)),
            pl.BlockSpec((1, 3, T), lambda b, i: (b, 0, i)),
        ],
        out_specs=(
            pl.BlockSpec((1, _K, T), lambda b, i: (b, 0, i)),
            pl.BlockSpec((1, _K, T), lambda b, i: (b, 0, i)),
            pl.BlockSpec((1, _K, T), lambda b, i: (b, 0, i)),
            pl.BlockSpec((1, 8, T), lambda b, i: (b, 0, i)),
        ),
        out_shape=out_shapes,
        interpret=interpret,
    )(src_pad, tgt_t)


def _assemble_body(g_ref, t_ref, patch_ref, ds_ref):
    x = g_ref[...]
    t = t_ref[...]
    R = x.shape[0]
    patch_ref[...] = x - t
    lanemod = lax.broadcasted_iota(jnp.int32, (R, _K * _D), 1) % _D
    dens = jnp.sum(jnp.where(lanemod == 3, x, 0.0), axis=1, keepdims=True)
    ds_ref[...] = jnp.concatenate([dens] * 8, axis=1)


def _run_assemble(g2d, t2d):
    NR = g2d.shape[0]
    R = 256
    return pl.pallas_call(
        _assemble_body,
        grid=(NR // R,),
        in_specs=[
            pl.BlockSpec((R, _K * _D), lambda i: (i, 0)),
            pl.BlockSpec((R, _K * _D), lambda i: (i, 0)),
        ],
        out_specs=(
            pl.BlockSpec((R, _K * _D), lambda i: (i, 0)),
            pl.BlockSpec((R, 8), lambda i: (i, 0)),
        ),
        out_shape=(
            jax.ShapeDtypeStruct((NR, _K * _D), jnp.float32),
            jax.ShapeDtypeStruct((NR, 8), jnp.float32),
        ),
    )(g2d, t2d)


def kernel(source_points, target_points, source_density, target_density):
    B, Ns, _ = source_points.shape
    _, Nt, _ = target_points.shape
    rad = jnp.full((B, 1, 1), _RADIUS, dtype=jnp.float32)

    src_pad = jnp.concatenate(
        [source_points, jnp.zeros((B, Ns, 5), jnp.float32)], axis=2)
    tgt_t = target_points.transpose(0, 2, 1)
    dist_t, pidx_t, gidx_t, stats_t = _run_topk(src_pad, tgt_t)
    dist = dist_t.transpose(0, 2, 1)
    pidx = pidx_t.transpose(0, 2, 1)

    mask_cnt = stats_t[:, 0, :, None]
    patches_size = stats_t[:, 1, :]

    src_n = source_points / (rad + 1e-06)
    tgt_n = target_points / (rad + 1e-06)
    table = jnp.concatenate(
        [src_n, source_density, jnp.zeros((B, Ns, _D - 4), jnp.float32)],
        axis=2).reshape(B * Ns, _D)
    table = jnp.concatenate(
        [table, jnp.zeros((_ZPAD, _D), jnp.float32)], axis=0)
    gidx_flat = gidx_t.transpose(0, 2, 1).reshape(-1)
    g = jnp.take(table, gidx_flat, axis=0)
    tpat = jnp.concatenate(
        [tgt_n, jnp.zeros((B, Nt, _D - 3), jnp.float32)], axis=2)
    tpat = jnp.broadcast_to(
        tpat[:, :, None, :], (B, Nt, _K, _D)).reshape(B * Nt, _K * _D)
    patches2d, ds8 = _run_assemble(g.reshape(B * Nt, _K * _D), tpat)
    patches = patches2d.reshape(B, Nt, _K, _D)[..., 0:3]
    weight_density = ds8[:, 0].reshape(B, Nt, 1) / mask_cnt

    bidx = jnp.arange(B, dtype=jnp.int32).reshape(B, 1, 1)
    bidx = jnp.broadcast_to(bidx, (B, Nt, _K))
    idx = jnp.stack([bidx, pidx], axis=-1)
    H = int(round(float(np.cbrt(Nt))))
    out_patches = patches.reshape(B, H, H, H, _K, 3)
    out_idx = idx.reshape(B, H, H, H, _K, 2)
    out_dist = dist.reshape(B, H, H, H, _K)
    return (out_patches, out_idx, out_dist, patches_size, rad, weight_density)
